# Initial kernel scaffold; baseline (speedup 1.0000x reference)
import jax
import jax.numpy as jnp
from jax.experimental import pallas as pl


def kernel(dens, points, down):
    raise NotImplementedError("write your pallas kernel here")


# speedup vs baseline: 2.2877x; 2.2877x over previous
import functools

import jax
import jax.numpy as jnp
from jax.experimental import pallas as pl
from jax.experimental.pallas import tpu as pltpu

_MIN_RADIUS = 8.0
_MAX_RADIUS = 96.0
_COST_POINT = 8.0
_COST_CLASS = 1.0
_EPS = 1e-08
_SCALE_WEIGHT = 0.02

_PIX_BLK = 1024


def _loss_body(npix, npts, gt_count, nblocks,
               scal_ref, pxf_ref, pyf_ref, bx_ref, by_ref, den_blk_ref,
               den_full_ref, out_ref):
    i = pl.program_id(0)

    px = pxf_ref[...]
    py = pyf_ref[...]
    bx = bx_ref[...]
    by = by_ref[...]

    dx = px - bx
    dy = py - by
    d2 = dx * dx + dy * dy
    m2 = jnp.min(d2, axis=1, keepdims=True)
    minc = jnp.sqrt(m2)

    t = (minc < _MIN_RADIUS).astype(jnp.float32)
    minc_cl = jnp.minimum(minc, _MAX_RADIUS)
    w = jnp.where(t > 0, _COST_POINT, _COST_CLASS * (minc_cl / _MAX_RADIUS))

    den_full = den_full_ref[...]
    dmax = jnp.max(jnp.maximum(den_full, 0.0))
    den = jnp.maximum(den_blk_ref[...], 0.0)
    p = jnp.where(dmax > 0, den / (dmax + _EPS), jnp.zeros_like(den))
    p = jnp.clip(p, 1e-07, 1.0 - 1e-07)
    bce = -(t * jnp.log(p) + (1.0 - t) * jnp.log(1.0 - p))

    partial = jnp.sum(w * bce)

    @pl.when(i == 0)
    def _():
        out_ref[0, 0] = 0.0

    out_ref[0, 0] += partial

    @pl.when(i == nblocks - 1)
    def _():
        down2 = scal_ref[0]
        pred_c = jnp.sum(den_full) / (down2 * down2)
        pen = _SCALE_WEIGHT * jnp.abs(pred_c - gt_count)
        out_ref[0, 0] = out_ref[0, 0] / npix + pen


def kernel(dens, points, down):
    down_f = jnp.asarray(down, dtype=jnp.float32)
    B = points.shape[0]
    assert B == 1
    den = dens[0, 0]
    H, W = den.shape
    npix = H * W
    n = points.shape[1]

    cols = jnp.arange(W, dtype=jnp.float32) * down_f + (down_f - 1.0) / 2.0
    rows = jnp.arange(H, dtype=jnp.float32) * down_f + (down_f - 1.0) / 2.0
    pxf = jnp.tile(cols, H).reshape(npix, 1)
    pyf = jnp.repeat(rows, W).reshape(npix, 1)

    w_in = W * down_f
    h_in = H * down_f
    sx = jnp.clip(points[0, :, 0].astype(jnp.float32), 0.0, w_in - 1.0)
    sy = jnp.clip(points[0, :, 1].astype(jnp.float32), 0.0, h_in - 1.0)
    npad = ((n + 127) // 128) * 128
    pad = jnp.full((npad - n,), 1e9, dtype=jnp.float32)
    bx = jnp.concatenate([sx, pad]).reshape(1, npad)
    by = jnp.concatenate([sy, pad]).reshape(1, npad)

    den_blk = den.reshape(npix, 1)
    den_full = den.reshape(1, npix)

    nblocks = npix // _PIX_BLK
    body = functools.partial(_loss_body, float(npix), n, float(n), nblocks)
    out = pl.pallas_call(
        body,
        grid=(nblocks,),
        in_specs=[
            pl.BlockSpec(memory_space=pltpu.SMEM),
            pl.BlockSpec((_PIX_BLK, 1), lambda i: (i, 0)),
            pl.BlockSpec((_PIX_BLK, 1), lambda i: (i, 0)),
            pl.BlockSpec((1, npad), lambda i: (0, 0)),
            pl.BlockSpec((1, npad), lambda i: (0, 0)),
            pl.BlockSpec((_PIX_BLK, 1), lambda i: (i, 0)),
            pl.BlockSpec((1, npix), lambda i: (0, 0)),
        ],
        out_specs=pl.BlockSpec((1, 1), lambda i: (0, 0),
                               memory_space=pltpu.SMEM),
        out_shape=jax.ShapeDtypeStruct((1, 1), jnp.float32),
        compiler_params=pltpu.CompilerParams(
            dimension_semantics=("arbitrary",),
        ),
    )(jnp.stack([down_f]), pxf, pyf, bx, by, den_blk, den_full)
    return out[0, 0]
